# baseline (speedup 1.0000x reference)
import functools

import jax
import jax.numpy as jnp
from jax import lax
from jax.experimental import pallas as pl
from jax.experimental.pallas import tpu as pltpu
from jax.experimental.pallas import tpu_sc as plsc

NS = 16
L = 16
NROWS = 32
NCOLS = 4096
N = NROWS * NCOLS
CHUNKS = NCOLS // L
GROUPS = NCOLS // 128

_mesh = plsc.VectorSubcoreMesh(
    core_axis_name="c", subcore_axis_name="s", num_cores=1, num_subcores=NS
)

_params = pltpu.CompilerParams(needs_layout_passes=False)


@functools.partial(
    pl.kernel,
    compiler_params=_params,
    out_type=(
        jax.ShapeDtypeStruct((N,), jnp.int32),
        jax.ShapeDtypeStruct((N,), jnp.int32),
    ),
    mesh=_mesh,
    scratch_types=[
        pltpu.VMEM((2, NCOLS), jnp.float32),
        pltpu.VMEM((NS, L), jnp.int32),
        pltpu.VMEM((L,), jnp.int32),
        pltpu.VMEM((NCOLS,), jnp.int32),
        pltpu.VMEM((NCOLS,), jnp.int32),
        pltpu.VMEM((NCOLS,), jnp.int32),
        pltpu.VMEM((NCOLS,), jnp.int32),
        pltpu.VMEM((GROUPS, 128), jnp.int32),
        pltpu.VMEM_SHARED((3 * NS, L), jnp.int32),
        pltpu.SemaphoreType.DMA,
    ],
)
def _nonzero_kernel(x_hbm, rows_hbm, cols_hbm, x2, cnts_v, pub_v,
                    cval0, cval1, rval0, rval1, dest_b, sh_counts, sem):
  s = lax.axis_index("s")
  li = jnp.arange(L, dtype=jnp.int32)
  r0 = 2 * s
  r1 = 2 * s + 1

  cp0 = pltpu.async_copy(x_hbm.at[r0], x2.at[0], sem)
  cp1 = pltpu.async_copy(x_hbm.at[r1], x2.at[1], sem)
  z16 = jnp.zeros((L,), jnp.int32)
  r0s = jnp.full((L,), r0, jnp.int32)
  r1s = jnp.full((L,), r1, jnp.int32)
  cp0.wait()

  def cstep0(k, a0):
    v0 = x2[0, pl.ds(k * L, L)]
    a0 = a0 + plsc.all_reduce_population_count(v0 != 0.0)
    ramp = k * L + li
    rval0[pl.ds(k * L, L)] = r0s
    cval0[pl.ds(k * L, L)] = ramp
    rval1[pl.ds(k * L, L)] = r1s
    cval1[pl.ds(k * L, L)] = ramp
    return a0

  a0 = lax.fori_loop(0, CHUNKS, cstep0, z16, unroll=8)
  cp1.wait()

  def cstep1(k, a1):
    v1 = x2[1, pl.ds(k * L, L)]
    return a1 + plsc.all_reduce_population_count(v1 != 0.0)

  a1 = lax.fori_loop(0, CHUNKS, cstep1, z16, unroll=8)
  pub_v[...] = jnp.where(li == 0, a0, jnp.where(li == 1, a1, 0))
  pltpu.sync_copy(pub_v, sh_counts.at[2 * NS + s])

  plsc.subcore_barrier()

  pltpu.sync_copy(sh_counts.at[pl.ds(2 * NS, NS)], cnts_v)
  c_lo = plsc.load_gather(cnts_v, [li >> 1, li & 1])
  hi = li + NS
  c_hi = plsc.load_gather(cnts_v, [hi >> 1, hi & 1])
  n_total = jnp.sum(c_lo) + jnp.sum(c_hi)

  def handle_row(w, which, rval_b, cval_b):
    off = jnp.sum(jnp.where(li < w, c_lo, 0)) + jnp.sum(
        jnp.where(hi < w, c_hi, 0))
    n_w = jnp.sum(jnp.where(li == w, c_lo, 0)) + jnp.sum(
        jnp.where(hi == w, c_hi, 0))
    zoff = n_total + w * NCOLS - off
    fast = jnp.logical_and(n_w == NCOLS, off % 8 == 0)

    @pl.when(fast)
    def _fast():
      o = pl.multiple_of(off, 8)
      cp_r = pltpu.async_copy(rval_b, rows_hbm.at[pl.ds(o, NCOLS)], sem)
      cp_c = pltpu.async_copy(cval_b, cols_hbm.at[pl.ds(o, NCOLS)], sem)
      cp_r.wait()
      cp_c.wait()

    @pl.when(jnp.logical_not(fast))
    def _general():
      def step(k, carry):
        off_nz, off_z = carry
        v = x2[which, pl.ds(k * L, L)]
        m = v != 0.0
        mi = m.astype(jnp.int32)
        excl = plsc.cumsum(mi) - mi
        dest = jnp.where(m, off_nz + excl, off_z + (li - excl))
        dest_b[k >> 3, pl.ds((k & 7) * L, L)] = dest
        rval_b[pl.ds(k * L, L)] = jnp.where(m, w, 0)
        cval_b[pl.ds(k * L, L)] = jnp.where(m, k * L + li, 0)
        pc = plsc.all_reduce_population_count(m)
        return off_nz + pc, off_z + (L - pc)

      init = (jnp.full((L,), off, jnp.int32),
              jnp.full((L,), zoff, jnp.int32))
      lax.fori_loop(0, CHUNKS, step, init)

      copies = []
      for g in range(GROUPS):
        copies.append(pltpu.async_copy(
            rval_b.at[pl.ds(g * 128, 128)],
            rows_hbm.at[dest_b.at[g]], sem))
        copies.append(pltpu.async_copy(
            cval_b.at[pl.ds(g * 128, 128)],
            cols_hbm.at[dest_b.at[g]], sem))
      for cp in copies:
        cp.wait()

  handle_row(r0, 0, rval0, cval0)
  handle_row(r1, 1, rval1, cval1)


def kernel(x):
  rows, cols = _nonzero_kernel(x)
  return rows, cols


# speedup vs baseline: 1.0126x; 1.0126x over previous
import functools

import jax
import jax.numpy as jnp
from jax import lax
from jax.experimental import pallas as pl
from jax.experimental.pallas import tpu as pltpu
from jax.experimental.pallas import tpu_sc as plsc

NC = 2
NS = 16
L = 16
NROWS = 32
NCOLS = 4096
N = NROWS * NCOLS
CHUNKS = NCOLS // L
GROUPS = NCOLS // 128

_mesh = plsc.VectorSubcoreMesh(
    core_axis_name="c", subcore_axis_name="s", num_cores=NC, num_subcores=NS
)

_params = pltpu.CompilerParams(needs_layout_passes=False)


@functools.partial(
    pl.kernel,
    compiler_params=_params,
    out_type=(
        jax.ShapeDtypeStruct((N,), jnp.int32),
        jax.ShapeDtypeStruct((N,), jnp.int32),
    ),
    mesh=_mesh,
    scratch_types=[
        pltpu.VMEM((2, NCOLS), jnp.float32),
        pltpu.VMEM((NCOLS,), jnp.float32),
        pltpu.VMEM((NS, L), jnp.int32),
        pltpu.VMEM((L,), jnp.int32),
        pltpu.VMEM((NCOLS,), jnp.int32),
        pltpu.VMEM((NCOLS,), jnp.int32),
        pltpu.VMEM((GROUPS, 128), jnp.int32),
        pltpu.VMEM_SHARED((3 * NS, L), jnp.int32),
        pltpu.SemaphoreType.DMA,
    ],
)
def _nonzero_kernel(x_hbm, rows_hbm, cols_hbm, x2, xrow, cnts_v, pub_v,
                    cval_b, rval_b, dest_b, sh_counts, sem):
  c = lax.axis_index("c")
  s = lax.axis_index("s")
  w = c * NS + s
  li = jnp.arange(L, dtype=jnp.int32)

  cp0 = pltpu.async_copy(x_hbm.at[2 * s], x2.at[0], sem)
  cp1 = pltpu.async_copy(x_hbm.at[2 * s + 1], x2.at[1], sem)
  wsplat = jnp.full((L,), w, jnp.int32)
  z16 = jnp.zeros((L,), jnp.int32)
  cp0.wait()

  def cstep0(k, a0):
    v0 = x2[0, pl.ds(k * L, L)]
    a0 = a0 + plsc.all_reduce_population_count(v0 != 0.0)
    rval_b[pl.ds(k * L, L)] = wsplat
    cval_b[pl.ds(k * L, L)] = k * L + li
    return a0

  a0 = lax.fori_loop(0, CHUNKS, cstep0, z16, unroll=8)
  cp1.wait()

  def cstep1(k, a1):
    v1 = x2[1, pl.ds(k * L, L)]
    return a1 + plsc.all_reduce_population_count(v1 != 0.0)

  a1 = lax.fori_loop(0, CHUNKS, cstep1, z16, unroll=8)
  pub_v[...] = jnp.where(li == 0, a0, jnp.where(li == 1, a1, 0))
  pltpu.sync_copy(pub_v, sh_counts.at[2 * NS + s])

  plsc.subcore_barrier()

  pltpu.sync_copy(sh_counts.at[pl.ds(2 * NS, NS)], cnts_v)
  c_lo = plsc.load_gather(cnts_v, [li >> 1, li & 1])
  hi = li + NS
  c_hi = plsc.load_gather(cnts_v, [hi >> 1, hi & 1])
  off = jnp.sum(jnp.where(li < w, c_lo, 0)) + jnp.sum(
      jnp.where(hi < w, c_hi, 0))
  n_w = jnp.sum(jnp.where(li == w, c_lo, 0)) + jnp.sum(
      jnp.where(hi == w, c_hi, 0))
  n_total = jnp.sum(c_lo) + jnp.sum(c_hi)
  zoff = n_total + w * NCOLS - off

  fast = jnp.logical_and(n_w == NCOLS, off % 8 == 0)

  @pl.when(fast)
  def _fast():
    o = pl.multiple_of(off, 8)
    cp_r = pltpu.async_copy(rval_b, rows_hbm.at[pl.ds(o, NCOLS)], sem)
    cp_c = pltpu.async_copy(cval_b, cols_hbm.at[pl.ds(o, NCOLS)], sem)
    cp_r.wait()
    cp_c.wait()

  @pl.when(jnp.logical_not(fast))
  def _general():
    pltpu.sync_copy(x_hbm.at[w], xrow)

    def step(k, carry):
      off_nz, off_z = carry
      v = xrow[pl.ds(k * L, L)]
      m = v != 0.0
      mi = m.astype(jnp.int32)
      excl = plsc.cumsum(mi) - mi
      dest = jnp.where(m, off_nz + excl, off_z + (li - excl))
      dest_b[k >> 3, pl.ds((k & 7) * L, L)] = dest
      rval_b[pl.ds(k * L, L)] = jnp.where(m, w, 0)
      cval_b[pl.ds(k * L, L)] = jnp.where(m, k * L + li, 0)
      pc = plsc.all_reduce_population_count(m)
      return off_nz + pc, off_z + (L - pc)

    init = (jnp.full((L,), off, jnp.int32), jnp.full((L,), zoff, jnp.int32))
    lax.fori_loop(0, CHUNKS, step, init)

    copies = []
    for g in range(GROUPS):
      copies.append(pltpu.async_copy(
          rval_b.at[pl.ds(g * 128, 128)], rows_hbm.at[dest_b.at[g]], sem))
      copies.append(pltpu.async_copy(
          cval_b.at[pl.ds(g * 128, 128)], cols_hbm.at[dest_b.at[g]], sem))
    for cp in copies:
      cp.wait()


def kernel(x):
  rows, cols = _nonzero_kernel(x)
  return rows, cols
